# Initial kernel scaffold; baseline (speedup 1.0000x reference)
import jax
import jax.numpy as jnp
from jax.experimental import pallas as pl


def kernel(traj_tokens, word_tokens, prompt_global, edge_index_temporal, edge_index_semantic, edge_index_global, traj_valid_len, params):
    raise NotImplementedError("write your pallas kernel here")


# speedup vs baseline: 1.6987x; 1.6987x over previous
import functools

import jax
import jax.numpy as jnp
from jax import lax
from jax.experimental import pallas as pl
from jax.experimental.pallas import tpu as pltpu
from jax.experimental.pallas import tpu_sc as plsc

D = 128
_EPS = 1e-5
_SQRT2 = 1.4142135623730951

_NC = 2
_NS = 16
_NPASS = 2


def _gelu(x):
  return 0.5 * x * (1.0 + lax.erf(x / _SQRT2))


def _ln(x, g, b):
  mu = jnp.mean(x, axis=-1, keepdims=True)
  xc = x - mu
  var = jnp.mean(xc * xc, axis=-1, keepdims=True)
  return xc * lax.rsqrt(var + _EPS) * g + b


def _dot(x, wt):
  return jnp.dot(x, wt, preferred_element_type=jnp.float32,
                 precision=lax.Precision.HIGHEST)


def _pre_body(x_ref, w1t_ref, b1_ref, w2t_ref, b2_ref, g_ref, b_ref, o_ref):
  x = x_ref[...]
  a = _gelu(_dot(x, w1t_ref[...]) + b1_ref[...])
  h = _dot(a, w2t_ref[...]) + b2_ref[...]
  o_ref[...] = _ln(h, g_ref[...], b_ref[...])


def _lnmm_body(x_ref, g_ref, b_ref, wt_ref, mb_ref, o_ref):
  h = _ln(x_ref[...], g_ref[...], b_ref[...])
  o_ref[...] = _dot(h, wt_ref[...]) + mb_ref[...]


def _post_body(x_ref, acc_ref, rdeg_ref, o_ref):
  o_ref[...] = x_ref[...] + _gelu(acc_ref[...] * rdeg_ref[...])


def _final_body(h_ref, g_ref, dg_ref, db_ref, wdt_ref, dbb_ref,
                p_ref, cwt_ref, cb_ref, o_ref):
  h = h_ref[...]
  d = _ln(g_ref[...] - h, dg_ref[...], db_ref[...])
  delta = _dot(d, wdt_ref[...]) + dbb_ref[...]
  ctx = _dot(p_ref[...], cwt_ref[...]) + cb_ref[...]
  o_ref[...] = h + delta + ctx


def _row_call(body, n_rows, r, n_in_blocked, n_full, out_rows=None):
  del n_full
  out_rows = n_rows if out_rows is None else out_rows
  grid = (out_rows // r,)
  def call(*args):
    specs = []
    for k, a in enumerate(args):
      if k < n_in_blocked:
        specs.append(pl.BlockSpec((r, D), lambda i: (i, 0)))
      else:
        shp = a.shape
        specs.append(pl.BlockSpec(shp, lambda i: tuple(0 for _ in shp)))
    return pl.pallas_call(
        body,
        grid=grid,
        in_specs=specs,
        out_specs=pl.BlockSpec((r, D), lambda i: (i, 0)),
        out_shape=jax.ShapeDtypeStruct((out_rows, D), jnp.float32),
    )(*args)
  return call


def _post_call(x, acc, rdeg, n_rows, r):
  grid = (n_rows // r,)
  return pl.pallas_call(
      _post_body,
      grid=grid,
      in_specs=[
          pl.BlockSpec((r, D), lambda i: (i, 0)),
          pl.BlockSpec((r, D), lambda i: (i, 0)),
          pl.BlockSpec((r, 1), lambda i: (i, 0)),
      ],
      out_specs=pl.BlockSpec((r, D), lambda i: (i, 0)),
      out_shape=jax.ShapeDtypeStruct((n_rows, D), jnp.float32),
  )(x, acc, rdeg)


def _make_edge_kernel(e_pad, q, blk):
  chunk = e_pad // _NS
  n_full = chunk // blk
  rem = chunk - n_full * blk
  assert rem % 128 == 0 and q % 128 == 0
  shrows = q + 128
  nzr = shrows // _NS
  wr = q // _NS
  n_pad = 4 * q
  zr = 64

  mesh = plsc.VectorSubcoreMesh(core_axis_name="c", subcore_axis_name="s",
                                num_cores=_NC, num_subcores=_NS)

  @functools.partial(
      pl.kernel,
      out_type=[
          jax.ShapeDtypeStruct((n_pad, D), jnp.float32),
          jax.ShapeDtypeStruct((n_pad,), jnp.float32),
      ],
      mesh=mesh,
      scratch_types=[
          pltpu.VMEM_SHARED((shrows, D), jnp.float32),
          pltpu.VMEM_SHARED((shrows,), jnp.float32),
          pltpu.VMEM((blk,), jnp.int32),
          pltpu.VMEM((blk,), jnp.int32),
          pltpu.VMEM((blk // 64, 64), jnp.int32),
          pltpu.VMEM((blk // 64, 64), jnp.int32),
          pltpu.VMEM((64, D), jnp.float32),
          pltpu.VMEM((64, D), jnp.float32),
          pltpu.VMEM((1024,), jnp.float32),
          pltpu.VMEM((64,), jnp.float32),
          pltpu.VMEM((1024,), jnp.float32),
          pltpu.SemaphoreType.DMA,
          pltpu.SemaphoreType.DMA,
      ],
  )
  def edge_kernel(m_hbm, src_hbm, dst_hbm, out_hbm, rdeg_hbm,
                  acc_sh, deg_sh, src_blk, dst_blk, srcrows, locrows,
                  rows0, rows1, zvec, ones, dbuf, g0, g1):
    c = lax.axis_index("c")
    s = lax.axis_index("s")
    zero16 = jnp.zeros((16,), jnp.float32)
    one16 = jnp.full((16,), 1.0, jnp.float32)

    def zv_body(i, carry):
      zvec[pl.ds(i * 16, 16)] = zero16
      return carry
    lax.fori_loop(0, 1024 // 16, zv_body, 0)
    for j in range(64 // 16):
      ones[pl.ds(j * 16, 16)] = one16

    chunk_base = s * chunk
    for p in range(_NPASS):
      base = (2 * p + c) * q
      dump = q + s * 8

      def zi_body(r_i, carry):
        for j in range(D // 16):
          rows0[r_i, pl.ds(j * 16, 16)] = zero16
          rows1[r_i, pl.ds(j * 16, 16)] = zero16
        return carry
      lax.fori_loop(0, zr, zi_body, 0)

      off = s * nzr
      nfz, rz = divmod(nzr, zr)
      for k in range(nfz):
        pltpu.sync_copy(rows0, acc_sh.at[pl.ds(off + k * zr, zr)])
      if rz:
        pltpu.sync_copy(rows0.at[pl.ds(0, rz)],
                        acc_sh.at[pl.ds(off + nfz * zr, rz)])
      pltpu.sync_copy(zvec.at[pl.ds(0, nzr)], deg_sh.at[pl.ds(off, nzr)])
      plsc.subcore_barrier()

      def do_block(bbase, nblk_sz):
        nch = nblk_sz // 64
        pltpu.sync_copy(src_hbm.at[pl.ds(bbase, nblk_sz)],
                        src_blk.at[pl.ds(0, nblk_sz)])
        pltpu.sync_copy(dst_hbm.at[pl.ds(bbase, nblk_sz)],
                        dst_blk.at[pl.ds(0, nblk_sz)])

        def tr_body(i, carry):
          sv = src_blk[pl.ds(i * 16, 16)]
          dv = dst_blk[pl.ds(i * 16, 16)]
          loc = dv - base
          mask = (loc >= 0) & (loc < q)
          clamped = jnp.where(mask, loc, dump + (sv & 7))
          locrows[i // 4, pl.ds((i % 4) * 16, 16)] = clamped
          srcrows[i // 4, pl.ds((i % 4) * 16, 16)] = sv
          return carry
        lax.fori_loop(0, nblk_sz // 16, tr_body, 0)

        def pair_body(t, carry):
          cp0 = pltpu.async_copy(m_hbm.at[srcrows.at[2 * t]], rows0, g0)
          cp1 = pltpu.async_copy(m_hbm.at[srcrows.at[2 * t + 1]], rows1, g1)
          cp0.wait()
          pltpu.sync_copy(rows0, acc_sh.at[locrows.at[2 * t]], add=True)
          pltpu.sync_copy(ones, deg_sh.at[locrows.at[2 * t]], add=True)
          cp1.wait()
          pltpu.sync_copy(rows1, acc_sh.at[locrows.at[2 * t + 1]], add=True)
          pltpu.sync_copy(ones, deg_sh.at[locrows.at[2 * t + 1]], add=True)
          return carry
        lax.fori_loop(0, nch // 2, pair_body, 0)
        if nch % 2:
          ch = nch - 1
          cp = pltpu.async_copy(m_hbm.at[srcrows.at[ch]], rows0, g0)
          cp.wait()
          pltpu.sync_copy(rows0, acc_sh.at[locrows.at[ch]], add=True)
          pltpu.sync_copy(ones, deg_sh.at[locrows.at[ch]], add=True)

      def full_body(b, carry):
        do_block(chunk_base + b * blk, blk)
        return carry
      lax.fori_loop(0, n_full, full_body, 0)
      if rem:
        do_block(chunk_base + n_full * blk, rem)
      plsc.subcore_barrier()

      woff = s * wr
      pltpu.sync_copy(acc_sh.at[pl.ds(woff, wr)],
                      out_hbm.at[pl.ds(base + woff, wr)])
      pltpu.sync_copy(deg_sh.at[pl.ds(woff, wr)], dbuf.at[pl.ds(0, wr)])

      def rc_body(i, carry):
        v = dbuf[pl.ds(i * 16, 16)]
        dbuf[pl.ds(i * 16, 16)] = 1.0 / jnp.maximum(v, 1.0)
        return carry
      lax.fori_loop(0, (wr + 15) // 16, rc_body, 0)
      pltpu.sync_copy(dbuf.at[pl.ds(0, wr)],
                      rdeg_hbm.at[pl.ds(base + woff, wr)])
      plsc.subcore_barrier()

  return edge_kernel, n_pad


_E_PAD_L = 501760
_Q_L = 12544
_NPAD_L = 4 * _Q_L
_E_PAD_G = 100352
_Q_G = 12672
_NPAD_G = 4 * _Q_G

_edge_cache = {}


def _edge_kernel(e_pad, q, blk=1024):
  key = (e_pad, q, blk)
  if key not in _edge_cache:
    _edge_cache[key] = _make_edge_kernel(e_pad, q, blk)
  return _edge_cache[key][0]


def _pad_edges(e, e_pad, oob, n_src):
  npad = e_pad - e.shape[1]
  fill = (jnp.arange(npad, dtype=jnp.int32) * 97) % n_src
  src = jnp.concatenate([e[0].astype(jnp.int32), fill])
  dst = jnp.pad(e[1].astype(jnp.int32), (0, npad), constant_values=oob)
  return src, dst


def kernel(traj_tokens, word_tokens, prompt_global, edge_index_temporal,
           edge_index_semantic, edge_index_global, traj_valid_len, params):
  p = params
  t = traj_tokens.shape[0]
  m_words = word_tokens.shape[0]
  n_glob = t + m_words

  scale = jnp.clip(p['traj_token_scale'], 0.0, 1.0)
  pre = _row_call(_pre_body, t, 400, 1, 6)
  h0 = pre(traj_tokens, p['proj_w1'].T, p['proj_b1'][None, :],
           p['proj_w2'].T, p['proj_b2'][None, :],
           (p['traj_ln_g'] * scale)[None, :],
           (p['traj_ln_b'] * scale)[None, :])

  e_local = jnp.concatenate([edge_index_temporal, edge_index_semantic], axis=1)
  src_l, dst_l = _pad_edges(e_local, _E_PAD_L, _NPAD_L, t)
  src_g, dst_g = _pad_edges(edge_index_global, _E_PAD_G, _NPAD_G, n_glob)

  lnmm = _row_call(_lnmm_body, t, 400, 1, 4)

  h = h0
  for lp in p['local']:
    msg = lnmm(h, lp['ln_g'][None, :], lp['ln_b'][None, :],
               lp['msg_w'].T, lp['msg_b'][None, :])
    acc, rdeg = _edge_kernel(_E_PAD_L, _Q_L)(msg, src_l, dst_l)
    h = _post_call(h, acc[:t], rdeg[:t, None], t, 400)

  prompt_inj = 0.3 * jax.nn.sigmoid(p['prompt_inj_logit'])
  traj_inj = 0.5 * jax.nn.sigmoid(p['traj_inj_logit'])
  ctx_scale = 0.3 * jax.nn.sigmoid(p['ctx_scale_logit'])

  word = word_tokens + prompt_inj * prompt_global[None, :]
  nodes = jnp.concatenate([h, word], axis=0)

  lp = p['global'][0]
  lnmm_g = _row_call(_lnmm_body, n_glob, 112, 1, 4)
  msg_g = lnmm_g(nodes, lp['ln_g'][None, :], lp['ln_b'][None, :],
                 lp['msg_w'].T, lp['msg_b'][None, :])
  acc_g, rdeg_g = _edge_kernel(_E_PAD_G, _Q_G)(msg_g, src_g, dst_g)
  g = _post_call(nodes, acc_g[:n_glob], rdeg_g[:n_glob, None], n_glob, 112)

  final = _row_call(_final_body, t, 400, 2, 7)
  out = final(h, g[:t], p['delta_ln_g'][None, :], p['delta_ln_b'][None, :],
              (traj_inj * p['delta_w']).T, (traj_inj * p['delta_b'])[None, :],
              prompt_global[None, :], (ctx_scale * p['ctx_w']).T,
              (ctx_scale * p['ctx_b'])[None, :])
  return out


# speedup vs baseline: 2.6991x; 1.5889x over previous
import functools

import jax
import jax.numpy as jnp
from jax import lax
from jax.experimental import pallas as pl
from jax.experimental.pallas import tpu as pltpu
from jax.experimental.pallas import tpu_sc as plsc

D = 128
_EPS = 1e-5
_SQRT2 = 1.4142135623730951

_NC = 2
_NS = 16
_NPASS = 2


def _gelu(x):
  return 0.5 * x * (1.0 + lax.erf(x / _SQRT2))


def _ln(x, g, b):
  mu = jnp.mean(x, axis=-1, keepdims=True)
  xc = x - mu
  var = jnp.mean(xc * xc, axis=-1, keepdims=True)
  return xc * lax.rsqrt(var + _EPS) * g + b


def _dot(x, wt):
  return jnp.dot(x, wt, preferred_element_type=jnp.float32,
                 precision=lax.Precision.HIGHEST)


def _msg_halves(h, g_ref, b_ref, wt_ref, mb_ref, m0_ref, m1_ref):
  m = _dot(_ln(h, g_ref[...], b_ref[...]), wt_ref[...]) + mb_ref[...]
  m0_ref[...] = m[:, :D // 2]
  m1_ref[...] = m[:, D // 2:]


def _pre_lnmm_body(x_ref, w1t_ref, b1_ref, w2t_ref, b2_ref, g_ref, b_ref,
                   lg_ref, lb_ref, wt_ref, mb_ref, h_ref, m0_ref, m1_ref):
  x = x_ref[...]
  a = _gelu(_dot(x, w1t_ref[...]) + b1_ref[...])
  h = _ln(_dot(a, w2t_ref[...]) + b2_ref[...], g_ref[...], b_ref[...])
  h_ref[...] = h
  _msg_halves(h, lg_ref, lb_ref, wt_ref, mb_ref, m0_ref, m1_ref)


def _gcn_update(x_ref, a0_ref, a1_ref, rdeg_ref):
  acc = jnp.concatenate([a0_ref[...], a1_ref[...]], axis=1)
  return x_ref[...] + _gelu(acc * rdeg_ref[...])


def _post_lnmm_body(x_ref, a0_ref, a1_ref, rdeg_ref,
                    lg_ref, lb_ref, wt_ref, mb_ref, h_ref, m0_ref, m1_ref):
  h = _gcn_update(x_ref, a0_ref, a1_ref, rdeg_ref)
  h_ref[...] = h
  _msg_halves(h, lg_ref, lb_ref, wt_ref, mb_ref, m0_ref, m1_ref)


def _lnmm_body(x_ref, g_ref, b_ref, wt_ref, mb_ref, o0_ref, o1_ref):
  _msg_halves(x_ref[...], g_ref, b_ref, wt_ref, mb_ref, o0_ref, o1_ref)


def _post_body(x_ref, a0_ref, a1_ref, rdeg_ref, o_ref):
  o_ref[...] = _gcn_update(x_ref, a0_ref, a1_ref, rdeg_ref)


def _post_final_body(h_ref, a0_ref, a1_ref, rdeg_ref, dg_ref, db_ref,
                     wdt_ref, dbb_ref, p_ref, cwt_ref, cb_ref, o_ref):
  h = h_ref[...]
  acc = jnp.concatenate([a0_ref[...], a1_ref[...]], axis=1)
  gd = _gelu(acc * rdeg_ref[...])
  d = _ln(gd, dg_ref[...], db_ref[...])
  delta = _dot(d, wdt_ref[...]) + dbb_ref[...]
  ctx = _dot(p_ref[...], cwt_ref[...]) + cb_ref[...]
  o_ref[...] = h + delta + ctx


def _pcall(body, n_rows, r, blocked_cols, out_cols):
  def call(*args):
    specs = []
    for k, a in enumerate(args):
      if k < len(blocked_cols):
        specs.append(pl.BlockSpec((r, blocked_cols[k]), lambda i: (i, 0)))
      else:
        shp = a.shape
        specs.append(pl.BlockSpec(shp, lambda i: tuple(0 for _ in shp)))
    return pl.pallas_call(
        body,
        grid=(n_rows // r,),
        in_specs=specs,
        out_specs=[pl.BlockSpec((r, c), lambda i: (i, 0)) for c in out_cols],
        out_shape=[jax.ShapeDtypeStruct((n_rows, c), jnp.float32)
                   for c in out_cols],
    )(*args)
  return call


def _make_edge_kernel(e_pad, q, blk):
  chunk = e_pad // _NS
  n_full = chunk // blk
  rem = chunk - n_full * blk
  assert rem % 128 == 0 and q % 128 == 0
  hd = D // 2
  shrows = q + 128
  nzr = shrows // _NS
  wr = q // _NS
  n_pad = 2 * q
  zr = 128

  mesh = plsc.VectorSubcoreMesh(core_axis_name="c", subcore_axis_name="s",
                                num_cores=_NC, num_subcores=_NS)

  @functools.partial(
      pl.kernel,
      out_type=[
          jax.ShapeDtypeStruct((n_pad, hd), jnp.float32),
          jax.ShapeDtypeStruct((n_pad, hd), jnp.float32),
          jax.ShapeDtypeStruct((n_pad,), jnp.float32),
      ],
      mesh=mesh,
      compiler_params=pltpu.CompilerParams(use_tc_tiling_on_sc=False),
      scratch_types=[
          pltpu.VMEM_SHARED((shrows, hd), jnp.float32),
          pltpu.VMEM_SHARED((shrows,), jnp.float32),
          pltpu.VMEM((2 * blk,), jnp.int32),
          pltpu.VMEM((blk // 128, 128), jnp.int32),
          pltpu.VMEM((blk // 128, 128), jnp.int32),
          pltpu.VMEM((128, hd), jnp.float32),
          pltpu.VMEM((128, hd), jnp.float32),
          pltpu.VMEM((1664,), jnp.float32),
          pltpu.VMEM((128,), jnp.float32),
          pltpu.VMEM((1664,), jnp.float32),
          pltpu.SemaphoreType.DMA,
          pltpu.SemaphoreType.DMA,
      ],
  )
  def edge_kernel(m0_hbm, m1_hbm, sd_hbm,
                  out0_hbm, out1_hbm, rdeg_hbm,
                  acc_sh, deg_sh, sd_blk, srcrows, locrows,
                  rows0, rows1, zvec, ones, dbuf, g0, g1):
    c = lax.axis_index("c")
    s = lax.axis_index("s")
    zero16 = jnp.zeros((16,), jnp.float32)
    one16 = jnp.full((16,), 1.0, jnp.float32)

    def zv_body(i, carry):
      zvec[pl.ds(i * 16, 16)] = zero16
      return carry
    lax.fori_loop(0, 1664 // 16, zv_body, 0)
    for j in range(128 // 16):
      ones[pl.ds(j * 16, 16)] = one16

    tile_base = s * (2 * chunk)
    base = c * q
    dump = q + s * 8
    off = s * nzr
    woff = s * wr

    for p in range(2):
      m_hbm = m0_hbm if p == 0 else m1_hbm
      out_hbm = out0_hbm if p == 0 else out1_hbm

      def zi_body(r_i, carry):
        for j in range(hd // 16):
          rows0[r_i, pl.ds(j * 16, 16)] = zero16
        return carry
      lax.fori_loop(0, zr, zi_body, 0)

      nfz, rz = divmod(nzr, zr)
      for k in range(nfz):
        pltpu.sync_copy(rows0, acc_sh.at[pl.ds(off + k * zr, zr)])
      if rz:
        pltpu.sync_copy(rows0.at[pl.ds(0, rz)],
                        acc_sh.at[pl.ds(off + nfz * zr, rz)])
      if p == 0:
        pltpu.sync_copy(zvec.at[pl.ds(0, nzr)], deg_sh.at[pl.ds(off, nzr)])
      plsc.subcore_barrier()

      rbufs = (rows0, rows1)
      gsems = (g0, g1)

      def do_block(bbase, nblk_sz):
        nch = nblk_sz // 128
        pltpu.sync_copy(sd_hbm.at[pl.ds(bbase, 2 * nblk_sz)],
                        sd_blk.at[pl.ds(0, 2 * nblk_sz)])

        def tr_body(i, carry):
          sv = sd_blk[pl.ds(i * 16, 16)]
          dv = sd_blk[pl.ds(nblk_sz + i * 16, 16)]
          loc = dv - base
          mask = (loc >= 0) & (loc < q)
          clamped = jnp.where(mask, loc, dump + (sv & 7))
          locrows[i // 8, pl.ds((i % 8) * 16, 16)] = clamped
          srcrows[i // 8, pl.ds((i % 8) * 16, 16)] = sv
          return carry
        lax.fori_loop(0, nblk_sz // 16, tr_body, 0)

        cps = {}
        for ch in range(min(2, nch)):
          cps[ch] = pltpu.async_copy(m_hbm.at[srcrows.at[ch]],
                                     rbufs[ch % 2], gsems[ch % 2])
        for ch in range(nch):
          cps[ch].wait()
          pltpu.sync_copy(rbufs[ch % 2], acc_sh.at[locrows.at[ch]], add=True)
          if p == 0:
            pltpu.sync_copy(ones, deg_sh.at[locrows.at[ch]], add=True)
          if ch + 2 < nch:
            cps[ch + 2] = pltpu.async_copy(m_hbm.at[srcrows.at[ch + 2]],
                                           rbufs[ch % 2], gsems[ch % 2])

      def full_body(b, carry):
        do_block(tile_base + b * 2 * blk, blk)
        return carry
      lax.fori_loop(0, n_full, full_body, 0)
      if rem:
        do_block(tile_base + n_full * 2 * blk, rem)
      plsc.subcore_barrier()

      pltpu.sync_copy(acc_sh.at[pl.ds(woff, wr)],
                      out_hbm.at[pl.ds(base + woff, wr)])
      if p == 0:
        pltpu.sync_copy(deg_sh.at[pl.ds(woff, wr)], dbuf.at[pl.ds(0, wr)])

        def rc_body(i, carry):
          v = dbuf[pl.ds(i * 16, 16)]
          dbuf[pl.ds(i * 16, 16)] = 1.0 / jnp.maximum(v, 1.0)
          return carry
        lax.fori_loop(0, (wr + 15) // 16, rc_body, 0)
        pltpu.sync_copy(dbuf.at[pl.ds(0, wr)],
                        rdeg_hbm.at[pl.ds(base + woff, wr)])
      plsc.subcore_barrier()

  return edge_kernel, n_pad


_E_PAD_L = 501760
_Q_L = 25088
_NPAD_L = 2 * _Q_L
_E_PAD_G = 100352
_Q_G = 25344
_NPAD_G = 2 * _Q_G

_edge_cache = {}


def _edge_kernel(e_pad, q, blk=1024):
  key = (e_pad, q, blk)
  if key not in _edge_cache:
    _edge_cache[key] = _make_edge_kernel(e_pad, q, blk)
  return _edge_cache[key][0]


def _pad_edges(e, e_pad, oob, n_src, blk=1024):
  npad = e_pad - e.shape[1]
  fill = (jnp.arange(npad, dtype=jnp.int32) * 97) % n_src
  src = jnp.concatenate([e[0].astype(jnp.int32), fill])
  dst = jnp.pad(e[1].astype(jnp.int32), (0, npad), constant_values=oob)
  chunk = e_pad // _NS
  n_full = chunk // blk
  rem = chunk - n_full * blk
  src_t = src.reshape(_NS, chunk)
  dst_t = dst.reshape(_NS, chunk)
  full = jnp.stack([src_t[:, :n_full * blk].reshape(_NS, n_full, blk),
                    dst_t[:, :n_full * blk].reshape(_NS, n_full, blk)],
                   axis=2).reshape(_NS, -1)
  parts = [full]
  if rem:
    parts.append(jnp.stack([src_t[:, n_full * blk:],
                            dst_t[:, n_full * blk:]], axis=1).reshape(_NS, -1))
  return jnp.concatenate(parts, axis=1).reshape(-1)


def kernel(traj_tokens, word_tokens, prompt_global, edge_index_temporal,
           edge_index_semantic, edge_index_global, traj_valid_len, params):
  p = params
  t = traj_tokens.shape[0]
  m_words = word_tokens.shape[0]
  n_glob = t + m_words

  hd = D // 2
  scale = jnp.clip(p['traj_token_scale'], 0.0, 1.0)
  lp1, lp2 = p['local']
  lpg = p['global'][0]

  e_local = jnp.concatenate([edge_index_temporal, edge_index_semantic], axis=1)
  sd_l = _pad_edges(e_local, _E_PAD_L, _NPAD_L, t)
  sd_g = _pad_edges(edge_index_global, _E_PAD_G, _NPAD_G, n_glob)

  h0, m0, m1 = _pcall(_pre_lnmm_body, t, 400, (D,), (D, hd, hd))(
      traj_tokens, p['proj_w1'].T, p['proj_b1'][None, :],
      p['proj_w2'].T, p['proj_b2'][None, :],
      (p['traj_ln_g'] * scale)[None, :], (p['traj_ln_b'] * scale)[None, :],
      lp1['ln_g'][None, :], lp1['ln_b'][None, :],
      lp1['msg_w'].T, lp1['msg_b'][None, :])
  a0, a1, rdeg = _edge_kernel(_E_PAD_L, _Q_L)(m0, m1, sd_l)

  h1, m0, m1 = _pcall(_post_lnmm_body, t, 400, (D, hd, hd, 1), (D, hd, hd))(
      h0, a0[:t], a1[:t], rdeg[:t, None],
      lp2['ln_g'][None, :], lp2['ln_b'][None, :],
      lp2['msg_w'].T, lp2['msg_b'][None, :])
  a0, a1, rdeg = _edge_kernel(_E_PAD_L, _Q_L)(m0, m1, sd_l)

  h2, = _pcall(_post_body, t, 400, (D, hd, hd, 1), (D,))(
      h1, a0[:t], a1[:t], rdeg[:t, None])

  prompt_inj = 0.3 * jax.nn.sigmoid(p['prompt_inj_logit'])
  traj_inj = 0.5 * jax.nn.sigmoid(p['traj_inj_logit'])
  ctx_scale = 0.3 * jax.nn.sigmoid(p['ctx_scale_logit'])

  word = word_tokens + prompt_inj * prompt_global[None, :]
  nodes = jnp.concatenate([h2, word], axis=0)

  mg0, mg1 = _pcall(_lnmm_body, n_glob, 112, (D,), (hd, hd))(
      nodes, lpg['ln_g'][None, :], lpg['ln_b'][None, :],
      lpg['msg_w'].T, lpg['msg_b'][None, :])
  ag0, ag1, rdeg_g = _edge_kernel(_E_PAD_G, _Q_G)(mg0, mg1, sd_g)

  out, = _pcall(_post_final_body, t, 400, (D, hd, hd, 1), (D,))(
      h2, ag0[:t], ag1[:t], rdeg_g[:t, None],
      p['delta_ln_g'][None, :], p['delta_ln_b'][None, :],
      (traj_inj * p['delta_w']).T, (traj_inj * p['delta_b'])[None, :],
      prompt_global[None, :], (ctx_scale * p['ctx_w']).T,
      (ctx_scale * p['ctx_b'])[None, :])
  return out


# speedup vs baseline: 2.9867x; 1.1066x over previous
import functools

import jax
import jax.numpy as jnp
from jax import lax
from jax.experimental import pallas as pl
from jax.experimental.pallas import tpu as pltpu
from jax.experimental.pallas import tpu_sc as plsc

D = 128
_EPS = 1e-5
_SQRT2 = 1.4142135623730951

_NC = 2
_NS = 16
_NPASS = 2


def _gelu(x):
  return 0.5 * x * (1.0 + lax.erf(x / _SQRT2))


def _ln(x, g, b):
  mu = jnp.mean(x, axis=-1, keepdims=True)
  xc = x - mu
  var = jnp.mean(xc * xc, axis=-1, keepdims=True)
  return xc * lax.rsqrt(var + _EPS) * g + b


def _dot(x, wt):
  return jnp.dot(x, wt, preferred_element_type=jnp.float32,
                 precision=lax.Precision.HIGHEST)


def _msg_halves(h, g_ref, b_ref, wt_ref, mb_ref, m0_ref, m1_ref):
  m = _dot(_ln(h, g_ref[...], b_ref[...]), wt_ref[...]) + mb_ref[...]
  m0_ref[...] = m[:, :D // 2]
  m1_ref[...] = m[:, D // 2:]


def _pre_lnmm_body(x_ref, w1t_ref, b1_ref, w2t_ref, b2_ref, g_ref, b_ref,
                   lg_ref, lb_ref, wt_ref, mb_ref, h_ref, m0_ref, m1_ref):
  x = x_ref[...]
  a = _gelu(_dot(x, w1t_ref[...]) + b1_ref[...])
  h = _ln(_dot(a, w2t_ref[...]) + b2_ref[...], g_ref[...], b_ref[...])
  h_ref[...] = h
  _msg_halves(h, lg_ref, lb_ref, wt_ref, mb_ref, m0_ref, m1_ref)


def _gcn_update(x_ref, a0_ref, a1_ref, rdeg_ref):
  acc = jnp.concatenate([a0_ref[...], a1_ref[...]], axis=1)
  return x_ref[...] + _gelu(acc * rdeg_ref[...])


def _post_lnmm_body(x_ref, a0_ref, a1_ref, rdeg_ref,
                    lg_ref, lb_ref, wt_ref, mb_ref, h_ref, m0_ref, m1_ref):
  h = _gcn_update(x_ref, a0_ref, a1_ref, rdeg_ref)
  h_ref[...] = h
  _msg_halves(h, lg_ref, lb_ref, wt_ref, mb_ref, m0_ref, m1_ref)


def _lnmm_body(x_ref, g_ref, b_ref, wt_ref, mb_ref, o0_ref, o1_ref):
  _msg_halves(x_ref[...], g_ref, b_ref, wt_ref, mb_ref, o0_ref, o1_ref)


def _post_body(x_ref, a0_ref, a1_ref, rdeg_ref, o_ref):
  o_ref[...] = _gcn_update(x_ref, a0_ref, a1_ref, rdeg_ref)


def _post_final_body(h_ref, a0_ref, a1_ref, rdeg_ref, dg_ref, db_ref,
                     wdt_ref, dbb_ref, p_ref, cwt_ref, cb_ref, o_ref):
  h = h_ref[...]
  acc = jnp.concatenate([a0_ref[...], a1_ref[...]], axis=1)
  gd = _gelu(acc * rdeg_ref[...])
  d = _ln(gd, dg_ref[...], db_ref[...])
  delta = _dot(d, wdt_ref[...]) + dbb_ref[...]
  ctx = _dot(p_ref[...], cwt_ref[...]) + cb_ref[...]
  o_ref[...] = h + delta + ctx


def _pcall(body, n_rows, r, blocked_cols, out_cols):
  def call(*args):
    specs = []
    for k, a in enumerate(args):
      if k < len(blocked_cols):
        specs.append(pl.BlockSpec((r, blocked_cols[k]), lambda i: (i, 0)))
      else:
        shp = a.shape
        specs.append(pl.BlockSpec(shp, lambda i: tuple(0 for _ in shp)))
    return pl.pallas_call(
        body,
        grid=(n_rows // r,),
        in_specs=specs,
        out_specs=[pl.BlockSpec((r, c), lambda i: (i, 0)) for c in out_cols],
        out_shape=[jax.ShapeDtypeStruct((n_rows, c), jnp.float32)
                   for c in out_cols],
    )(*args)
  return call


def _make_edge_kernel(e_pad, q, blk):
  chunk = e_pad // _NS
  n_full = chunk // blk
  rem = chunk - n_full * blk
  assert rem % 128 == 0 and q % 128 == 0
  hd = D // 2
  shrows = q + 128
  nzr = shrows // _NS
  wr = q // _NS
  n_pad = 2 * q
  zr = 64

  mesh = plsc.VectorSubcoreMesh(core_axis_name="c", subcore_axis_name="s",
                                num_cores=_NC, num_subcores=_NS)

  @functools.partial(
      pl.kernel,
      out_type=[
          jax.ShapeDtypeStruct((n_pad, hd), jnp.float32),
          jax.ShapeDtypeStruct((n_pad, hd), jnp.float32),
          jax.ShapeDtypeStruct((n_pad,), jnp.float32),
      ],
      mesh=mesh,
      compiler_params=pltpu.CompilerParams(use_tc_tiling_on_sc=False),
      scratch_types=[
          pltpu.VMEM_SHARED((shrows, hd), jnp.float32),
          pltpu.VMEM_SHARED((shrows,), jnp.float32),
          pltpu.VMEM((2 * blk,), jnp.int32),
          pltpu.VMEM((blk // 64, 64), jnp.int32),
          pltpu.VMEM((blk // 64, 64), jnp.int32),
          pltpu.VMEM((64, hd), jnp.float32),
          pltpu.VMEM((64, hd), jnp.float32),
          pltpu.VMEM((64, hd), jnp.float32),
          pltpu.VMEM((64, hd), jnp.float32),
          pltpu.VMEM((1664,), jnp.float32),
          pltpu.VMEM((64,), jnp.float32),
          pltpu.VMEM((1664,), jnp.float32),
          pltpu.SemaphoreType.DMA,
          pltpu.SemaphoreType.DMA,
          pltpu.SemaphoreType.DMA,
          pltpu.SemaphoreType.DMA,
          pltpu.SemaphoreType.DMA,
          pltpu.SemaphoreType.DMA,
          pltpu.SemaphoreType.DMA,
          pltpu.SemaphoreType.DMA,
          pltpu.SemaphoreType.DMA,
      ],
  )
  def edge_kernel(m0_hbm, m1_hbm, sd_hbm,
                  out0_hbm, out1_hbm, rdeg_hbm,
                  acc_sh, deg_sh, sd_blk, srcrows, locrows,
                  rows0, rows1, rows2, rows3, zvec, ones, dbuf,
                  g0, g1, g2, g3, a0, a1, a2, a3, dsem):
    c = lax.axis_index("c")
    s = lax.axis_index("s")
    zero16 = jnp.zeros((16,), jnp.float32)
    one16 = jnp.full((16,), 1.0, jnp.float32)

    def zv_body(i, carry):
      zvec[pl.ds(i * 16, 16)] = zero16
      return carry
    lax.fori_loop(0, 1664 // 16, zv_body, 0)
    for j in range(64 // 16):
      ones[pl.ds(j * 16, 16)] = one16

    tile_base = s * (2 * chunk)
    base = c * q
    dump = q + s * 8
    off = s * nzr
    woff = s * wr

    for p in range(2):
      m_hbm = m0_hbm if p == 0 else m1_hbm
      out_hbm = out0_hbm if p == 0 else out1_hbm

      def zi_body(r_i, carry):
        for j in range(hd // 16):
          rows0[r_i, pl.ds(j * 16, 16)] = zero16
        return carry
      lax.fori_loop(0, zr, zi_body, 0)

      nfz, rz = divmod(nzr, zr)
      zcs = [pltpu.async_copy(rows0, acc_sh.at[pl.ds(off + k * zr, zr)], dsem)
             for k in range(nfz)]
      if rz:
        zcs.append(pltpu.async_copy(rows0.at[pl.ds(0, rz)],
                                    acc_sh.at[pl.ds(off + nfz * zr, rz)],
                                    dsem))
      if p == 0:
        zcs.append(pltpu.async_copy(zvec.at[pl.ds(0, nzr)],
                                    deg_sh.at[pl.ds(off, nzr)], dsem))
      for z in zcs:
        z.wait()
      plsc.subcore_barrier()

      rbufs = (rows0, rows1, rows2, rows3)
      gsems = (g0, g1, g2, g3)
      asems = (a0, a1, a2, a3)

      def do_block(bbase, nblk_sz):
        nch = nblk_sz // 64
        pltpu.sync_copy(sd_hbm.at[pl.ds(bbase, 2 * nblk_sz)],
                        sd_blk.at[pl.ds(0, 2 * nblk_sz)])

        def tr_body(i, carry):
          sv = sd_blk[pl.ds(i * 16, 16)]
          dv = sd_blk[pl.ds(nblk_sz + i * 16, 16)]
          loc = dv - base
          mask = (loc >= 0) & (loc < q)
          clamped = jnp.where(mask, loc, dump + (sv & 7))
          locrows[i // 4, pl.ds((i % 4) * 16, 16)] = clamped
          srcrows[i // 4, pl.ds((i % 4) * 16, 16)] = sv
          return carry
        lax.fori_loop(0, nblk_sz // 16, tr_body, 0)

        dscs = []
        if p == 0:
          for ch in range(nch):
            dscs.append(pltpu.async_copy(ones, deg_sh.at[locrows.at[ch]],
                                         dsem, add=True))

        cps, scs = {}, {}
        for ch in range(min(3, nch)):
          cps[ch] = pltpu.async_copy(m_hbm.at[srcrows.at[ch]],
                                     rbufs[ch % 4], gsems[ch % 4])
        for ch in range(nch):
          cps[ch].wait()
          scs[ch] = pltpu.async_copy(rbufs[ch % 4],
                                     acc_sh.at[locrows.at[ch]],
                                     asems[ch % 4], add=True)
          nxt = ch + 3
          if nxt < nch:
            if nxt - 4 >= 0:
              scs[nxt - 4].wait()
            cps[nxt] = pltpu.async_copy(m_hbm.at[srcrows.at[nxt]],
                                        rbufs[nxt % 4], gsems[nxt % 4])
        for ch in range(max(0, nch - 4), nch):
          if ch >= 0 and ch in scs:
            scs[ch].wait()
        for d in dscs:
          d.wait()

      def full_body(b, carry):
        do_block(tile_base + b * 2 * blk, blk)
        return carry
      lax.fori_loop(0, n_full, full_body, 0)
      if rem:
        do_block(tile_base + n_full * 2 * blk, rem)
      plsc.subcore_barrier()

      pltpu.sync_copy(acc_sh.at[pl.ds(woff, wr)],
                      out_hbm.at[pl.ds(base + woff, wr)])
      if p == 0:
        pltpu.sync_copy(deg_sh.at[pl.ds(woff, wr)], dbuf.at[pl.ds(0, wr)])

        def rc_body(i, carry):
          v = dbuf[pl.ds(i * 16, 16)]
          dbuf[pl.ds(i * 16, 16)] = 1.0 / jnp.maximum(v, 1.0)
          return carry
        lax.fori_loop(0, (wr + 15) // 16, rc_body, 0)
        pltpu.sync_copy(dbuf.at[pl.ds(0, wr)],
                        rdeg_hbm.at[pl.ds(base + woff, wr)])
      plsc.subcore_barrier()

  return edge_kernel, n_pad


_E_PAD_L = 501760
_Q_L = 25088
_NPAD_L = 2 * _Q_L
_E_PAD_G = 100352
_Q_G = 25344
_NPAD_G = 2 * _Q_G

_edge_cache = {}


def _edge_kernel(e_pad, q, blk=1024):
  key = (e_pad, q, blk)
  if key not in _edge_cache:
    _edge_cache[key] = _make_edge_kernel(e_pad, q, blk)
  return _edge_cache[key][0]


def _pad_edges(e, e_pad, oob, n_src, blk=1024):
  npad = e_pad - e.shape[1]
  fill = (jnp.arange(npad, dtype=jnp.int32) * 97) % n_src
  src = jnp.concatenate([e[0].astype(jnp.int32), fill])
  dst = jnp.pad(e[1].astype(jnp.int32), (0, npad), constant_values=oob)
  chunk = e_pad // _NS
  n_full = chunk // blk
  rem = chunk - n_full * blk
  src_t = src.reshape(_NS, chunk)
  dst_t = dst.reshape(_NS, chunk)
  full = jnp.stack([src_t[:, :n_full * blk].reshape(_NS, n_full, blk),
                    dst_t[:, :n_full * blk].reshape(_NS, n_full, blk)],
                   axis=2).reshape(_NS, -1)
  parts = [full]
  if rem:
    parts.append(jnp.stack([src_t[:, n_full * blk:],
                            dst_t[:, n_full * blk:]], axis=1).reshape(_NS, -1))
  return jnp.concatenate(parts, axis=1).reshape(-1)


def kernel(traj_tokens, word_tokens, prompt_global, edge_index_temporal,
           edge_index_semantic, edge_index_global, traj_valid_len, params):
  p = params
  t = traj_tokens.shape[0]
  m_words = word_tokens.shape[0]
  n_glob = t + m_words

  hd = D // 2
  scale = jnp.clip(p['traj_token_scale'], 0.0, 1.0)
  lp1, lp2 = p['local']
  lpg = p['global'][0]

  e_local = jnp.concatenate([edge_index_temporal, edge_index_semantic], axis=1)
  sd_l = _pad_edges(e_local, _E_PAD_L, _NPAD_L, t)
  sd_g = _pad_edges(edge_index_global, _E_PAD_G, _NPAD_G, n_glob)

  h0, m0, m1 = _pcall(_pre_lnmm_body, t, 1000, (D,), (D, hd, hd))(
      traj_tokens, p['proj_w1'].T, p['proj_b1'][None, :],
      p['proj_w2'].T, p['proj_b2'][None, :],
      (p['traj_ln_g'] * scale)[None, :], (p['traj_ln_b'] * scale)[None, :],
      lp1['ln_g'][None, :], lp1['ln_b'][None, :],
      lp1['msg_w'].T, lp1['msg_b'][None, :])
  a0, a1, rdeg = _edge_kernel(_E_PAD_L, _Q_L)(m0, m1, sd_l)

  h1, m0, m1 = _pcall(_post_lnmm_body, t, 1000, (D, hd, hd, 1), (D, hd, hd))(
      h0, a0, a1, rdeg[:, None],
      lp2['ln_g'][None, :], lp2['ln_b'][None, :],
      lp2['msg_w'].T, lp2['msg_b'][None, :])
  a0, a1, rdeg = _edge_kernel(_E_PAD_L, _Q_L)(m0, m1, sd_l)

  h2, = _pcall(_post_body, t, 1000, (D, hd, hd, 1), (D,))(
      h1, a0, a1, rdeg[:, None])

  prompt_inj = 0.3 * jax.nn.sigmoid(p['prompt_inj_logit'])
  traj_inj = 0.5 * jax.nn.sigmoid(p['traj_inj_logit'])
  ctx_scale = 0.3 * jax.nn.sigmoid(p['ctx_scale_logit'])

  word = word_tokens + prompt_inj * prompt_global[None, :]
  n_gpad = _NPAD_G
  nodes = jnp.concatenate(
      [h2, word, jnp.zeros((n_gpad - n_glob, D), jnp.float32)], axis=0)

  mg0, mg1 = _pcall(_lnmm_body, n_gpad, 512, (D,), (hd, hd))(
      nodes, lpg['ln_g'][None, :], lpg['ln_b'][None, :],
      lpg['msg_w'].T, lpg['msg_b'][None, :])
  ag0, ag1, rdeg_g = _edge_kernel(_E_PAD_G, _Q_G)(mg0, mg1, sd_g)

  out, = _pcall(_post_final_body, t, 1000, (D, hd, hd, 1), (D,))(
      h2, ag0, ag1, rdeg_g[:, None],
      p['delta_ln_g'][None, :], p['delta_ln_b'][None, :],
      (traj_inj * p['delta_w']).T, (traj_inj * p['delta_b'])[None, :],
      prompt_global[None, :], (ctx_scale * p['ctx_w']).T,
      (ctx_scale * p['ctx_b'])[None, :])
  return out
